# Initial kernel scaffold; baseline (speedup 1.0000x reference)
import jax
import jax.numpy as jnp
from jax.experimental import pallas as pl


def kernel(x, adj_indices, adj_values, W1, b1, W2, b2):
    raise NotImplementedError("write your pallas kernel here")


# speedup vs baseline: 14.5872x; 14.5872x over previous
import functools

import jax
import jax.numpy as jnp
from jax import lax
from jax.experimental import pallas as pl
from jax.experimental.pallas import tpu as pltpu
from jax.experimental.pallas import tpu_sc as plsc

NREL = 25
NND = 10000
NEDGE = 320000
NFEAT = 128
FD = 16
NC, NS = 2, 16
G = 80
CHUNK = 2000
NG = CHUNK // G
EPT = NEDGE // NS
NCHUNK = EPT // CHUNK
RPT = NND // NS
RPC = 13
BM = 2000


def _spmm_body(sup_hbm, rows_hbm, cols_hbm, vals_hbm, out_hbm,
               cols_v, rows_v, vals_v, gath_v, zero_v, acc_sh, gsem, ssem):
    c = lax.axis_index("c")
    s = lax.axis_index("s")
    rbase = s * RPT

    def _zb(r, carry):
        zero_v[r, :] = jnp.zeros((FD,), jnp.float32)
        return carry
    lax.fori_loop(0, RPT, _zb, 0)

    def _rel(k, carry):
        i = c * RPC + k

        @pl.when(i < NREL)
        def _():
            pltpu.sync_copy(zero_v, acc_sh.at[pl.ds(rbase, RPT)])
            plsc.subcore_barrier()

            def _chunk(j, carry2):
                goff = s * (EPT // G) + j * NG
                eoff = pl.multiple_of(s * EPT + j * CHUNK, 8)
                pltpu.sync_copy(cols_hbm.at[i, pl.ds(goff, NG), :], cols_v)
                pltpu.sync_copy(rows_hbm.at[i, pl.ds(goff, NG), :], rows_v)
                pltpu.sync_copy(vals_hbm.at[i, pl.ds(eoff, CHUNK)], vals_v)

                off = i * NND

                def _addoff(q, carry3):
                    g = q // (G // 16)
                    t = q % (G // 16)
                    sl = pl.ds(t * 16, 16)
                    cols_v[g, sl] = cols_v[g, sl] + off
                    return carry3
                lax.fori_loop(0, NG * (G // 16), _addoff, 0, unroll=5)

                def _fire_g(g, carry3):
                    gdst = pl.ds(pl.multiple_of(g * G, 8), G)
                    pltpu.async_copy(sup_hbm.at[cols_v.at[g]], gath_v.at[gdst],
                                     gsem)
                    return carry3
                lax.fori_loop(0, NG, _fire_g, 0)

                def _drain_g(g, carry3):
                    gdst = pl.ds(pl.multiple_of(g * G, 8), G)
                    pltpu.make_async_copy(sup_hbm.at[cols_v.at[g]],
                                          gath_v.at[gdst], gsem).wait()
                    return carry3
                lax.fori_loop(0, NG, _drain_g, 0)

                def _scale(q, carry3):
                    base = q * 16
                    vv = vals_v[pl.ds(base, 16)]
                    for t in range(16):
                        e = base + t
                        gath_v[e, :] = gath_v[e, :] * vv[t]
                    return carry3
                lax.fori_loop(0, CHUNK // 16, _scale, 0)

                def _fire_s(g, carry3):
                    gsrc = pl.ds(pl.multiple_of(g * G, 8), G)
                    pltpu.async_copy(gath_v.at[gsrc], acc_sh.at[rows_v.at[g]],
                                     ssem, add=True)
                    return carry3
                lax.fori_loop(0, NG, _fire_s, 0)

                def _drain_s(g, carry3):
                    gsrc = pl.ds(pl.multiple_of(g * G, 8), G)
                    pltpu.make_async_copy(gath_v.at[gsrc],
                                          acc_sh.at[rows_v.at[g]],
                                          ssem).wait()
                    return carry3
                lax.fori_loop(0, NG, _drain_s, 0)
                return carry2
            lax.fori_loop(0, NCHUNK, _chunk, 0)

            plsc.subcore_barrier()
            pltpu.sync_copy(acc_sh.at[pl.ds(rbase, RPT)],
                            out_hbm.at[i, pl.ds(rbase, RPT), :])
        return carry
    lax.fori_loop(0, RPC, _rel, 0)


@functools.lru_cache(maxsize=None)
def _make_spmm():
    return pl.kernel(
        _spmm_body,
        out_type=jax.ShapeDtypeStruct((NREL, NND, FD), jnp.float32),
        mesh=plsc.VectorSubcoreMesh(core_axis_name="c", subcore_axis_name="s",
                                    num_cores=NC, num_subcores=NS),
        compiler_params=pltpu.CompilerParams(use_tc_tiling_on_sc=False),
        scratch_types=[
            pltpu.VMEM((NG, G), jnp.int32),
            pltpu.VMEM((NG, G), jnp.int32),
            pltpu.VMEM((CHUNK,), jnp.float32),
            pltpu.VMEM((CHUNK, FD), jnp.float32),
            pltpu.VMEM((RPT, FD), jnp.float32),
            pltpu.VMEM_SHARED((NND, FD), jnp.float32),
            pltpu.SemaphoreType.DMA,
            pltpu.SemaphoreType.DMA,
        ],
    )


def _spmm(sup_flat, adj_rows, adj_cols, vals):
    return _make_spmm()(sup_flat, adj_rows, adj_cols, vals)


def _mm1_body(x_ref, w_ref, o_ref):
    o_ref[...] = jnp.dot(x_ref[...], w_ref[...],
                         preferred_element_type=jnp.float32,
                         precision=lax.Precision.HIGHEST)


def _mm2_body(a_ref, b_ref, w_ref, o_ref):
    h = jnp.maximum(a_ref[...] + b_ref[...], 0.0)
    o_ref[...] = jnp.dot(h, w_ref[...], preferred_element_type=jnp.float32,
                         precision=lax.Precision.HIGHEST)


def _max_body(a_ref, b_ref, o_ref):
    r = pl.program_id(1)
    v = jnp.maximum(a_ref[0] + b_ref[0], 0.0)

    @pl.when(r == 0)
    def _():
        o_ref[...] = v

    @pl.when(r > 0)
    def _():
        o_ref[...] = jnp.maximum(o_ref[...], v)


def _mm1(x, w):
    return pl.pallas_call(
        _mm1_body,
        grid=(NND // BM,),
        in_specs=[pl.BlockSpec((BM, NFEAT), lambda m: (m, 0)),
                  pl.BlockSpec((NFEAT, NREL * FD), lambda m: (0, 0))],
        out_specs=pl.BlockSpec((BM, NREL * FD), lambda m: (m, 0)),
        out_shape=jax.ShapeDtypeStruct((NND, NREL * FD), jnp.float32),
    )(x, w)


def _mm2(a, b, w):
    return pl.pallas_call(
        _mm2_body,
        grid=(NND // BM,),
        in_specs=[pl.BlockSpec((BM, NREL * FD), lambda m: (m, 0)),
                  pl.BlockSpec((1, NREL * FD), lambda m: (0, 0)),
                  pl.BlockSpec((NREL * FD, NREL * FD), lambda m: (0, 0))],
        out_specs=pl.BlockSpec((BM, NREL * FD), lambda m: (m, 0)),
        out_shape=jax.ShapeDtypeStruct((NND, NREL * FD), jnp.float32),
    )(a, b, w)


def _maxpool(a, b2):
    return pl.pallas_call(
        _max_body,
        grid=(NND // BM, NREL),
        in_specs=[pl.BlockSpec((1, BM, FD), lambda m, r: (r, m, 0)),
                  pl.BlockSpec((1, 1, FD), lambda m, r: (r, 0, 0))],
        out_specs=pl.BlockSpec((BM, FD), lambda m, r: (m, 0)),
        out_shape=jax.ShapeDtypeStruct((NND, FD), jnp.float32),
    )(a, b2)


def kernel(x, adj_indices, adj_values, W1, b1, W2, b2):
    adj_indices = adj_indices.astype(jnp.int32)
    adj_rows = adj_indices[:, 0, :].reshape(NREL, NEDGE // G, G)
    adj_cols = adj_indices[:, 1, :].reshape(NREL, NEDGE // G, G)
    vals = adj_values.astype(jnp.float32)

    W1f = W1.transpose(1, 0, 2).reshape(NFEAT, NREL * FD)
    W2f = W2.transpose(1, 0, 2).reshape(NREL * FD, NREL * FD)
    b1f = b1.reshape(1, NREL * FD)
    b2r = b2.reshape(NREL, 1, FD)

    s1f = _mm1(x, W1f)
    sup1 = s1f.reshape(NND, NREL, FD).transpose(1, 0, 2)
    agg1 = _spmm(sup1.reshape(NREL * NND, FD), adj_rows, adj_cols, vals)
    agg1f = agg1.transpose(1, 0, 2).reshape(NND, NREL * FD)
    s2f = _mm2(agg1f, b1f, W2f)
    sup2 = s2f.reshape(NND, NREL, FD).transpose(1, 0, 2)
    agg2 = _spmm(sup2.reshape(NREL * NND, FD), adj_rows, adj_cols, vals)
    return _maxpool(agg2, b2r)


# speedup vs baseline: 16.8861x; 1.1576x over previous
import functools

import jax
import jax.numpy as jnp
from jax import lax
from jax.experimental import pallas as pl
from jax.experimental.pallas import tpu as pltpu
from jax.experimental.pallas import tpu_sc as plsc

NREL = 25
NND = 10000
NEDGE = 320000
NFEAT = 128
FD = 16
NC, NS = 2, 16
G = 80
CHUNK = 2000
NG = CHUNK // G
EPT = NEDGE // NS
NCHUNK = EPT // CHUNK
GPT = EPT // G
RPT = NND // NS
RPC = 13
BM = 2000


def _spmm_body(sup_hbm, rows_hbm, cols_hbm, vals_hbm, out_hbm,
               cols_v, rows_v, vals_v, gath_v, zero_v, acc_sh,
               lsem, gsem, ssem):
    c = lax.axis_index("c")
    s = lax.axis_index("s")
    rbase = s * RPT

    def _zb(r, carry):
        zero_v[r, :] = jnp.zeros((FD,), jnp.float32)
        return carry
    lax.fori_loop(0, RPT, _zb, 0)

    def _fire_loads(i, j, b):
        goff = s * GPT + j * NG
        eoff = pl.multiple_of(s * EPT + j * CHUNK, 8)
        pltpu.async_copy(cols_hbm.at[i, pl.ds(goff, NG), :], cols_v.at[b],
                         lsem)
        pltpu.async_copy(rows_hbm.at[i, pl.ds(goff, NG), :], rows_v.at[b],
                         lsem)
        pltpu.async_copy(vals_hbm.at[i, pl.ds(eoff, CHUNK)], vals_v.at[b],
                         lsem)

    def _drain_loads(i, j, b):
        goff = s * GPT + j * NG
        eoff = pl.multiple_of(s * EPT + j * CHUNK, 8)
        pltpu.make_async_copy(cols_hbm.at[i, pl.ds(goff, NG), :],
                              cols_v.at[b], lsem).wait()
        pltpu.make_async_copy(rows_hbm.at[i, pl.ds(goff, NG), :],
                              rows_v.at[b], lsem).wait()
        pltpu.make_async_copy(vals_hbm.at[i, pl.ds(eoff, CHUNK)],
                              vals_v.at[b], lsem).wait()

    def _addoff(i, b):
        off = i * NND

        def _body(q, carry):
            g = q // (G // 16)
            t = q % (G // 16)
            sl = pl.ds(t * 16, 16)
            cols_v[b, g, sl] = cols_v[b, g, sl] + off
            return carry
        lax.fori_loop(0, NG * (G // 16), _body, 0, unroll=5)

    def _fire_gathers(b):
        def _body(g, carry):
            gdst = pl.ds(pl.multiple_of(g * G, 8), G)
            pltpu.async_copy(sup_hbm.at[cols_v.at[b, g]],
                             gath_v.at[b].at[gdst], gsem)
            return carry
        lax.fori_loop(0, NG, _body, 0)

    def _drain_gathers(b):
        def _body(g, carry):
            gdst = pl.ds(pl.multiple_of(g * G, 8), G)
            pltpu.make_async_copy(sup_hbm.at[cols_v.at[b, g]],
                                  gath_v.at[b].at[gdst], gsem).wait()
            return carry
        lax.fori_loop(0, NG, _body, 0)

    def _scale(b):
        def _body(q, carry):
            base = q * 16
            vv = vals_v[b, pl.ds(base, 16)]
            for t in range(16):
                e = base + t
                gath_v[b, e, :] = gath_v[b, e, :] * vv[t]
            return carry
        lax.fori_loop(0, CHUNK // 16, _body, 0)

    def _fire_scatters(b):
        def _body(g, carry):
            gsrc = pl.ds(pl.multiple_of(g * G, 8), G)
            pltpu.async_copy(gath_v.at[b].at[gsrc],
                             acc_sh.at[rows_v.at[b, g]], ssem, add=True)
            return carry
        lax.fori_loop(0, NG, _body, 0)

    def _drain_scatters(b):
        def _body(g, carry):
            gsrc = pl.ds(pl.multiple_of(g * G, 8), G)
            pltpu.make_async_copy(gath_v.at[b].at[gsrc],
                                  acc_sh.at[rows_v.at[b, g]], ssem).wait()
            return carry
        lax.fori_loop(0, NG, _body, 0)

    def _rel(k, carry):
        i = c * RPC + k

        @pl.when(i < NREL)
        def _():
            pltpu.sync_copy(zero_v, acc_sh.at[pl.ds(rbase, RPT)])
            plsc.subcore_barrier()

            _fire_loads(i, 0, 0)
            _drain_loads(i, 0, 0)
            _addoff(i, 0)
            _fire_gathers(0)
            for j in range(NCHUNK):
                b = j % 2
                if j + 1 < NCHUNK:
                    if j >= 1:
                        _drain_scatters(1 - b)
                    _fire_loads(i, j + 1, 1 - b)
                _drain_gathers(b)
                if j + 1 < NCHUNK:
                    _drain_loads(i, j + 1, 1 - b)
                    _addoff(i, 1 - b)
                    _fire_gathers(1 - b)
                _scale(b)
                _fire_scatters(b)
            _drain_scatters(0)
            _drain_scatters(1)

            plsc.subcore_barrier()
            pltpu.sync_copy(acc_sh.at[pl.ds(rbase, RPT)],
                            out_hbm.at[i, pl.ds(rbase, RPT), :])
        return carry
    lax.fori_loop(0, RPC, _rel, 0)


@functools.lru_cache(maxsize=None)
def _make_spmm():
    return pl.kernel(
        _spmm_body,
        out_type=jax.ShapeDtypeStruct((NREL, NND, FD), jnp.float32),
        mesh=plsc.VectorSubcoreMesh(core_axis_name="c", subcore_axis_name="s",
                                    num_cores=NC, num_subcores=NS),
        compiler_params=pltpu.CompilerParams(use_tc_tiling_on_sc=False),
        scratch_types=[
            pltpu.VMEM((2, NG, G), jnp.int32),
            pltpu.VMEM((2, NG, G), jnp.int32),
            pltpu.VMEM((2, CHUNK), jnp.float32),
            pltpu.VMEM((2, CHUNK, FD), jnp.float32),
            pltpu.VMEM((RPT, FD), jnp.float32),
            pltpu.VMEM_SHARED((NND, FD), jnp.float32),
            pltpu.SemaphoreType.DMA,
            pltpu.SemaphoreType.DMA,
            pltpu.SemaphoreType.DMA,
        ],
    )


def _spmm(sup_flat, adj_rows, adj_cols, vals):
    return _make_spmm()(sup_flat, adj_rows, adj_cols, vals)


def _mm1_body(x_ref, w_ref, o_ref):
    o_ref[0] = jnp.dot(x_ref[...], w_ref[0],
                       preferred_element_type=jnp.float32,
                       precision=lax.Precision.HIGHEST)


BMT = 1000


def _relu_tr_body(a_ref, b_ref, o_ref):
    h = jnp.maximum(a_ref[...] + b_ref[...], 0.0)
    o_ref[...] = jnp.swapaxes(h, 0, 1).reshape(BMT, NREL * FD)


def _max_body(a_ref, b_ref, o_ref):
    r = pl.program_id(1)
    v = jnp.maximum(a_ref[0] + b_ref[0], 0.0)

    @pl.when(r == 0)
    def _():
        o_ref[...] = v

    @pl.when(r > 0)
    def _():
        o_ref[...] = jnp.maximum(o_ref[...], v)


def _mm1(x, w):
    return pl.pallas_call(
        _mm1_body,
        grid=(NND // BM, NREL),
        in_specs=[pl.BlockSpec((BM, NFEAT), lambda m, r: (m, 0)),
                  pl.BlockSpec((1, NFEAT, FD), lambda m, r: (r, 0, 0))],
        out_specs=pl.BlockSpec((1, BM, FD), lambda m, r: (r, m, 0)),
        out_shape=jax.ShapeDtypeStruct((NREL, NND, FD), jnp.float32),
    )(x, w)


def _relu_tr(a, b):
    return pl.pallas_call(
        _relu_tr_body,
        grid=(NND // BMT,),
        in_specs=[pl.BlockSpec((NREL, BMT, FD), lambda m: (0, m, 0)),
                  pl.BlockSpec((NREL, 1, FD), lambda m: (0, 0, 0))],
        out_specs=pl.BlockSpec((BMT, NREL * FD), lambda m: (m, 0)),
        out_shape=jax.ShapeDtypeStruct((NND, NREL * FD), jnp.float32),
    )(a, b)


def _mm1b(h, w):
    return pl.pallas_call(
        _mm1_body,
        grid=(NND // BM, NREL),
        in_specs=[pl.BlockSpec((BM, NREL * FD), lambda m, r: (m, 0)),
                  pl.BlockSpec((1, NREL * FD, FD), lambda m, r: (r, 0, 0))],
        out_specs=pl.BlockSpec((1, BM, FD), lambda m, r: (r, m, 0)),
        out_shape=jax.ShapeDtypeStruct((NREL, NND, FD), jnp.float32),
    )(h, w)


def _maxpool(a, b2):
    return pl.pallas_call(
        _max_body,
        grid=(NND // BM, NREL),
        in_specs=[pl.BlockSpec((1, BM, FD), lambda m, r: (r, m, 0)),
                  pl.BlockSpec((1, 1, FD), lambda m, r: (r, 0, 0))],
        out_specs=pl.BlockSpec((BM, FD), lambda m, r: (m, 0)),
        out_shape=jax.ShapeDtypeStruct((NND, FD), jnp.float32),
    )(a, b2)


def kernel(x, adj_indices, adj_values, W1, b1, W2, b2):
    adj_indices = adj_indices.astype(jnp.int32)
    adj_rows = adj_indices[:, 0, :].reshape(NREL, NEDGE // G, G)
    adj_cols = adj_indices[:, 1, :].reshape(NREL, NEDGE // G, G)
    vals = adj_values.astype(jnp.float32)

    b1r = b1.reshape(NREL, 1, FD)
    b2r = b2.reshape(NREL, 1, FD)

    sup1 = _mm1(x, W1)
    agg1 = _spmm(sup1.reshape(NREL * NND, FD), adj_rows, adj_cols, vals)
    htr = _relu_tr(agg1, b1r)
    sup2 = _mm1b(htr, W2)
    agg2 = _spmm(sup2.reshape(NREL * NND, FD), adj_rows, adj_cols, vals)
    return _maxpool(agg2, b2r)


# speedup vs baseline: 21.6439x; 1.2818x over previous
import functools

import jax
import jax.numpy as jnp
from jax import lax
from jax.experimental import pallas as pl
from jax.experimental.pallas import tpu as pltpu
from jax.experimental.pallas import tpu_sc as plsc

NREL = 25
NND = 10000
NEDGE = 320000
NFEAT = 128
FD = 16
NC, NS = 2, 16
G = 80
CHUNK = 2000
NG = CHUNK // G
EPT = NEDGE // NS
NCHUNK = EPT // CHUNK
GPT = EPT // G
RPT = NND // NS
RPC = 13
BM = 2000


def _spmm_pipeline(sup_hbm, rows_hbm, cols_hbm, vals_hbm,
                   cols_v, rows_v, vals_v, gath_v, acc_sh,
                   lsem, gsem, ssem, s, i):
    def _fire_loads(j, b):
        goff = s * GPT + j * NG
        eoff = pl.multiple_of(s * EPT + j * CHUNK, 8)
        pltpu.async_copy(cols_hbm.at[i, pl.ds(goff, NG), :], cols_v.at[b],
                         lsem)
        pltpu.async_copy(rows_hbm.at[i, pl.ds(goff, NG), :], rows_v.at[b],
                         lsem)
        pltpu.async_copy(vals_hbm.at[i, pl.ds(eoff, CHUNK)], vals_v.at[b],
                         lsem)

    def _drain_loads(j, b):
        goff = s * GPT + j * NG
        eoff = pl.multiple_of(s * EPT + j * CHUNK, 8)
        pltpu.make_async_copy(cols_hbm.at[i, pl.ds(goff, NG), :],
                              cols_v.at[b], lsem).wait()
        pltpu.make_async_copy(rows_hbm.at[i, pl.ds(goff, NG), :],
                              rows_v.at[b], lsem).wait()
        pltpu.make_async_copy(vals_hbm.at[i, pl.ds(eoff, CHUNK)],
                              vals_v.at[b], lsem).wait()

    def _addoff(b):
        def _body(q, carry):
            g = q // (G // 16)
            t = q % (G // 16)
            sl = pl.ds(t * 16, 16)
            cols_v[b, g, sl] = cols_v[b, g, sl] * NREL + i
            return carry
        lax.fori_loop(0, NG * (G // 16), _body, 0, unroll=5)

    def _fire_gathers(b):
        def _body(g, carry):
            gdst = pl.ds(pl.multiple_of(g * G, 8), G)
            pltpu.async_copy(sup_hbm.at[cols_v.at[b, g]],
                             gath_v.at[b].at[gdst], gsem)
            return carry
        lax.fori_loop(0, NG, _body, 0)

    def _drain_gathers(b):
        def _body(g, carry):
            gdst = pl.ds(pl.multiple_of(g * G, 8), G)
            pltpu.make_async_copy(sup_hbm.at[cols_v.at[b, g]],
                                  gath_v.at[b].at[gdst], gsem).wait()
            return carry
        lax.fori_loop(0, NG, _body, 0)

    def _scale(b):
        def _body(q, carry):
            base = q * 16
            vv = vals_v[b, pl.ds(base, 16)]
            for t in range(16):
                e = base + t
                gath_v[b, e, :] = gath_v[b, e, :] * vv[t]
            return carry
        lax.fori_loop(0, CHUNK // 16, _body, 0)

    def _fire_scatters(b):
        def _body(g, carry):
            gsrc = pl.ds(pl.multiple_of(g * G, 8), G)
            pltpu.async_copy(gath_v.at[b].at[gsrc],
                             acc_sh.at[rows_v.at[b, g]], ssem, add=True)
            return carry
        lax.fori_loop(0, NG, _body, 0)

    def _drain_scatters(b):
        def _body(g, carry):
            gsrc = pl.ds(pl.multiple_of(g * G, 8), G)
            pltpu.make_async_copy(gath_v.at[b].at[gsrc],
                                  acc_sh.at[rows_v.at[b, g]], ssem).wait()
            return carry
        lax.fori_loop(0, NG, _body, 0)

    _fire_loads(0, 0)
    _drain_loads(0, 0)
    _addoff(0)
    _fire_gathers(0)
    for j in range(NCHUNK):
        b = j % 2
        if j + 1 < NCHUNK:
            if j >= 1:
                _drain_scatters(1 - b)
            _fire_loads(j + 1, 1 - b)
        _drain_gathers(b)
        if j + 1 < NCHUNK:
            _drain_loads(j + 1, 1 - b)
            _addoff(1 - b)
            _fire_gathers(1 - b)
        _scale(b)
        _fire_scatters(b)
    _drain_scatters(0)
    _drain_scatters(1)


def _zero_buf(zero_v):
    def _zb(r, carry):
        zero_v[r, :] = jnp.zeros((FD,), jnp.float32)
        return carry
    lax.fori_loop(0, RPT, _zb, 0)


def _spmm1_body(sup_hbm, rows_hbm, cols_hbm, vals_hbm, b_hbm, h_hbm,
                cols_v, rows_v, vals_v, gath_v, zero_v, tmp_v, bv_v,
                acc_sh, lsem, gsem, ssem):
    c = lax.axis_index("c")
    s = lax.axis_index("s")
    rbase = s * RPT
    _zero_buf(zero_v)
    pltpu.sync_copy(b_hbm, bv_v)

    def _rel(k, carry):
        i = c * RPC + k

        @pl.when(i < NREL)
        def _():
            pltpu.sync_copy(zero_v, acc_sh.at[pl.ds(rbase, RPT)])
            plsc.subcore_barrier()
            _spmm_pipeline(sup_hbm, rows_hbm, cols_hbm, vals_hbm,
                           cols_v, rows_v, vals_v, gath_v, acc_sh,
                           lsem, gsem, ssem, s, i)
            plsc.subcore_barrier()
            pltpu.sync_copy(acc_sh.at[pl.ds(rbase, RPT)], tmp_v)
            bv = bv_v[i, :]

            def _br(r, carry2):
                tmp_v[r, :] = jnp.maximum(tmp_v[r, :] + bv, 0.0)
                return carry2
            lax.fori_loop(0, RPT, _br, 0)
            pltpu.sync_copy(tmp_v, h_hbm.at[pl.ds(rbase, RPT), i, :])
        return carry
    lax.fori_loop(0, RPC, _rel, 0)


def _spmm2_body(sup_hbm, rows_hbm, cols_hbm, vals_hbm, b_hbm, pmax_hbm,
                cols_v, rows_v, vals_v, gath_v, zero_v, tmp_v, bv_v,
                runmax_v, acc_sh, lsem, gsem, ssem):
    c = lax.axis_index("c")
    s = lax.axis_index("s")
    rbase = s * RPT
    _zero_buf(zero_v)
    pltpu.sync_copy(b_hbm, bv_v)

    def _rel(k, carry):
        i = c * RPC + k

        @pl.when(i < NREL)
        def _():
            pltpu.sync_copy(zero_v, acc_sh.at[pl.ds(rbase, RPT)])
            plsc.subcore_barrier()
            _spmm_pipeline(sup_hbm, rows_hbm, cols_hbm, vals_hbm,
                           cols_v, rows_v, vals_v, gath_v, acc_sh,
                           lsem, gsem, ssem, s, i)
            plsc.subcore_barrier()
            pltpu.sync_copy(acc_sh.at[pl.ds(rbase, RPT)], tmp_v)
            bv = bv_v[i, :]

            @pl.when(k == 0)
            def _():
                def _init(r, carry2):
                    runmax_v[r, :] = tmp_v[r, :] + bv
                    return carry2
                lax.fori_loop(0, RPT, _init, 0)

            @pl.when(k > 0)
            def _():
                def _merge(r, carry2):
                    runmax_v[r, :] = jnp.maximum(runmax_v[r, :],
                                                 tmp_v[r, :] + bv)
                    return carry2
                lax.fori_loop(0, RPT, _merge, 0)
        return carry
    lax.fori_loop(0, RPC, _rel, 0)
    pltpu.sync_copy(runmax_v, pmax_hbm.at[c, pl.ds(rbase, RPT), :])


_SPMM_SCRATCH = (
    pltpu.VMEM((2, NG, G), jnp.int32),
    pltpu.VMEM((2, NG, G), jnp.int32),
    pltpu.VMEM((2, CHUNK), jnp.float32),
    pltpu.VMEM((2, CHUNK, FD), jnp.float32),
    pltpu.VMEM((RPT, FD), jnp.float32),
    pltpu.VMEM((RPT, FD), jnp.float32),
    pltpu.VMEM((NREL, FD), jnp.float32),
)


def _sc_mesh():
    return plsc.VectorSubcoreMesh(core_axis_name="c", subcore_axis_name="s",
                                  num_cores=NC, num_subcores=NS)


@functools.lru_cache(maxsize=None)
def _make_spmm1():
    return pl.kernel(
        _spmm1_body,
        out_type=jax.ShapeDtypeStruct((NND, NREL, FD), jnp.float32),
        mesh=_sc_mesh(),
        compiler_params=pltpu.CompilerParams(use_tc_tiling_on_sc=False),
        scratch_types=[
            *_SPMM_SCRATCH,
            pltpu.VMEM_SHARED((NND, FD), jnp.float32),
            pltpu.SemaphoreType.DMA,
            pltpu.SemaphoreType.DMA,
            pltpu.SemaphoreType.DMA,
        ],
    )


@functools.lru_cache(maxsize=None)
def _make_spmm2():
    return pl.kernel(
        _spmm2_body,
        out_type=jax.ShapeDtypeStruct((NC, NND, FD), jnp.float32),
        mesh=_sc_mesh(),
        compiler_params=pltpu.CompilerParams(use_tc_tiling_on_sc=False),
        scratch_types=[
            *_SPMM_SCRATCH,
            pltpu.VMEM((RPT, FD), jnp.float32),
            pltpu.VMEM_SHARED((NND, FD), jnp.float32),
            pltpu.SemaphoreType.DMA,
            pltpu.SemaphoreType.DMA,
            pltpu.SemaphoreType.DMA,
        ],
    )


def _mm_body(x_ref, w_ref, o_ref):
    o_ref[...] = jnp.dot(x_ref[...], w_ref[...],
                         preferred_element_type=jnp.float32,
                         precision=lax.Precision.HIGHEST)


def _maxfin_body(p_ref, o_ref):
    o_ref[...] = jnp.maximum(jnp.maximum(p_ref[0], p_ref[1]), 0.0)


def _mm(x, w):
    m, k = x.shape
    n = w.shape[1]
    return pl.pallas_call(
        _mm_body,
        grid=(m // BM,),
        in_specs=[pl.BlockSpec((BM, k), lambda mm_: (mm_, 0)),
                  pl.BlockSpec((k, n), lambda mm_: (0, 0))],
        out_specs=pl.BlockSpec((BM, n), lambda mm_: (mm_, 0)),
        out_shape=jax.ShapeDtypeStruct((m, n), jnp.float32),
    )(x, w)


def _maxfin(p):
    return pl.pallas_call(
        _maxfin_body,
        grid=(NND // BM,),
        in_specs=[pl.BlockSpec((NC, BM, FD), lambda m: (0, m, 0))],
        out_specs=pl.BlockSpec((BM, FD), lambda m: (m, 0)),
        out_shape=jax.ShapeDtypeStruct((NND, FD), jnp.float32),
    )(p)


def kernel(x, adj_indices, adj_values, W1, b1, W2, b2):
    adj_indices = adj_indices.astype(jnp.int32)
    adj_rows = adj_indices[:, 0, :].reshape(NREL, NEDGE // G, G)
    adj_cols = adj_indices[:, 1, :].reshape(NREL, NEDGE // G, G)
    vals = adj_values.astype(jnp.float32)

    W1f = W1.transpose(1, 0, 2).reshape(NFEAT, NREL * FD)
    W2f = W2.transpose(1, 0, 2).reshape(NREL * FD, NREL * FD)

    s1f = _mm(x, W1f)
    h3 = _make_spmm1()(s1f.reshape(NND * NREL, FD), adj_rows, adj_cols,
                       vals, b1)
    s2f = _mm(h3.reshape(NND, NREL * FD), W2f)
    pmax = _make_spmm2()(s2f.reshape(NND * NREL, FD), adj_rows, adj_cols,
                         vals, b2)
    return _maxfin(pmax)


# speedup vs baseline: 24.3754x; 1.1262x over previous
import functools

import jax
import jax.numpy as jnp
from jax import lax
from jax.experimental import pallas as pl
from jax.experimental.pallas import tpu as pltpu
from jax.experimental.pallas import tpu_sc as plsc

NREL = 25
NND = 10000
NEDGE = 320000
NFEAT = 128
FD = 16
NC, NS = 2, 16
G = 80
CHUNK = 2000
NG = CHUNK // G
EPT = NEDGE // NS
NCHUNK = EPT // CHUNK
GPT = EPT // G
RPT = NND // NS
RPC = 13
BM = 2000


def _spmm_pipeline(sup_hbm, rows_hbm, cols_hbm, vals_hbm,
                   cols_v, rows_v, vals_v, gath_v, acc_sh,
                   lsem, gsem, ssem, s, i):
    def _fire_loads(j, b):
        goff = s * GPT + j * NG
        eoff = pl.multiple_of(s * EPT + j * CHUNK, 8)
        pltpu.async_copy(cols_hbm.at[i, pl.ds(goff, NG), :], cols_v.at[b],
                         lsem)
        pltpu.async_copy(rows_hbm.at[i, pl.ds(goff, NG), :], rows_v.at[b],
                         lsem)
        pltpu.async_copy(vals_hbm.at[i, pl.ds(eoff, CHUNK)], vals_v.at[b],
                         lsem)

    def _drain_loads(j, b):
        goff = s * GPT + j * NG
        eoff = pl.multiple_of(s * EPT + j * CHUNK, 8)
        pltpu.make_async_copy(cols_hbm.at[i, pl.ds(goff, NG), :],
                              cols_v.at[b], lsem).wait()
        pltpu.make_async_copy(rows_hbm.at[i, pl.ds(goff, NG), :],
                              rows_v.at[b], lsem).wait()
        pltpu.make_async_copy(vals_hbm.at[i, pl.ds(eoff, CHUNK)],
                              vals_v.at[b], lsem).wait()

    def _addoff(b):
        def _body(g, carry):
            for t in range(G // 16):
                sl = pl.ds(t * 16, 16)
                cols_v[b, g, sl] = cols_v[b, g, sl] * NREL + i
            return carry
        lax.fori_loop(0, NG, _body, 0, unroll=2)

    def _fire_gathers(b):
        def _body(g, carry):
            gdst = pl.ds(pl.multiple_of(g * G, 8), G)
            pltpu.async_copy(sup_hbm.at[cols_v.at[b, g]],
                             gath_v.at[b].at[gdst], gsem)
            return carry
        lax.fori_loop(0, NG, _body, 0)

    def _drain_gathers(b):
        def _body(g, carry):
            gdst = pl.ds(pl.multiple_of(g * G, 8), G)
            pltpu.make_async_copy(sup_hbm.at[cols_v.at[b, g]],
                                  gath_v.at[b].at[gdst], gsem).wait()
            return carry
        lax.fori_loop(0, NG, _body, 0)

    def _scale(b):
        @plsc.parallel_loop(0, CHUNK, step=16)
        def _body(base):
            vv = vals_v[b, pl.ds(base, 16)]
            for t in range(16):
                e = base + t
                gath_v[b, e, :] = gath_v[b, e, :] * vv[t]

    def _fire_scatters(b):
        def _body(g, carry):
            gsrc = pl.ds(pl.multiple_of(g * G, 8), G)
            pltpu.async_copy(gath_v.at[b].at[gsrc],
                             acc_sh.at[rows_v.at[b, g]], ssem, add=True)
            return carry
        lax.fori_loop(0, NG, _body, 0)

    def _drain_scatters(b):
        def _body(g, carry):
            gsrc = pl.ds(pl.multiple_of(g * G, 8), G)
            pltpu.make_async_copy(gath_v.at[b].at[gsrc],
                                  acc_sh.at[rows_v.at[b, g]], ssem).wait()
            return carry
        lax.fori_loop(0, NG, _body, 0)

    _fire_loads(0, 0)
    _drain_loads(0, 0)
    _addoff(0)
    _fire_gathers(0)
    for j in range(NCHUNK):
        b = j % 2
        if j + 1 < NCHUNK:
            if j >= 1:
                _drain_scatters(1 - b)
            _fire_loads(j + 1, 1 - b)
        _drain_gathers(b)
        if j + 1 < NCHUNK:
            _drain_loads(j + 1, 1 - b)
            _addoff(1 - b)
            _fire_gathers(1 - b)
        _scale(b)
        _fire_scatters(b)
    _drain_scatters(0)
    _drain_scatters(1)


def _zero_buf(zero_v):
    def _zb(r, carry):
        zero_v[r, :] = jnp.zeros((FD,), jnp.float32)
        return carry
    lax.fori_loop(0, RPT, _zb, 0)


def _spmm1_body(sup_hbm, rows_hbm, cols_hbm, vals_hbm, b_hbm, h_hbm,
                cols_v, rows_v, vals_v, gath_v, zero_v, tmp_v, bv_v,
                acc_sh, lsem, gsem, ssem):
    c = lax.axis_index("c")
    s = lax.axis_index("s")
    rbase = s * RPT
    _zero_buf(zero_v)
    pltpu.sync_copy(b_hbm, bv_v)

    def _rel(k, carry):
        i = c * RPC + k

        @pl.when(i < NREL)
        def _():
            pltpu.sync_copy(zero_v, acc_sh.at[pl.ds(rbase, RPT)])
            plsc.subcore_barrier()
            _spmm_pipeline(sup_hbm, rows_hbm, cols_hbm, vals_hbm,
                           cols_v, rows_v, vals_v, gath_v, acc_sh,
                           lsem, gsem, ssem, s, i)
            plsc.subcore_barrier()
            pltpu.sync_copy(acc_sh.at[pl.ds(rbase, RPT)], tmp_v)
            bv = bv_v[i, :]

            def _br(r, carry2):
                tmp_v[r, :] = jnp.maximum(tmp_v[r, :] + bv, 0.0)
                return carry2
            lax.fori_loop(0, RPT, _br, 0)
            pltpu.sync_copy(tmp_v, h_hbm.at[pl.ds(rbase, RPT), i, :])
        return carry
    lax.fori_loop(0, RPC, _rel, 0)


def _spmm2_body(sup_hbm, rows_hbm, cols_hbm, vals_hbm, b_hbm, pmax_hbm,
                cols_v, rows_v, vals_v, gath_v, zero_v, tmp_v, bv_v,
                runmax_v, acc_sh, lsem, gsem, ssem):
    c = lax.axis_index("c")
    s = lax.axis_index("s")
    rbase = s * RPT
    _zero_buf(zero_v)
    pltpu.sync_copy(b_hbm, bv_v)

    def _rel(k, carry):
        i = c * RPC + k

        @pl.when(i < NREL)
        def _():
            pltpu.sync_copy(zero_v, acc_sh.at[pl.ds(rbase, RPT)])
            plsc.subcore_barrier()
            _spmm_pipeline(sup_hbm, rows_hbm, cols_hbm, vals_hbm,
                           cols_v, rows_v, vals_v, gath_v, acc_sh,
                           lsem, gsem, ssem, s, i)
            plsc.subcore_barrier()
            pltpu.sync_copy(acc_sh.at[pl.ds(rbase, RPT)], tmp_v)
            bv = bv_v[i, :]

            @pl.when(k == 0)
            def _():
                def _init(r, carry2):
                    runmax_v[r, :] = tmp_v[r, :] + bv
                    return carry2
                lax.fori_loop(0, RPT, _init, 0)

            @pl.when(k > 0)
            def _():
                def _merge(r, carry2):
                    runmax_v[r, :] = jnp.maximum(runmax_v[r, :],
                                                 tmp_v[r, :] + bv)
                    return carry2
                lax.fori_loop(0, RPT, _merge, 0)
        return carry
    lax.fori_loop(0, RPC, _rel, 0)
    pltpu.sync_copy(runmax_v, pmax_hbm.at[c, pl.ds(rbase, RPT), :])


_SPMM_SCRATCH = (
    pltpu.VMEM((2, NG, G), jnp.int32),
    pltpu.VMEM((2, NG, G), jnp.int32),
    pltpu.VMEM((2, CHUNK), jnp.float32),
    pltpu.VMEM((2, CHUNK, FD), jnp.float32),
    pltpu.VMEM((RPT, FD), jnp.float32),
    pltpu.VMEM((RPT, FD), jnp.float32),
    pltpu.VMEM((NREL, FD), jnp.float32),
)


def _sc_mesh():
    return plsc.VectorSubcoreMesh(core_axis_name="c", subcore_axis_name="s",
                                  num_cores=NC, num_subcores=NS)


@functools.lru_cache(maxsize=None)
def _make_spmm1():
    return pl.kernel(
        _spmm1_body,
        out_type=jax.ShapeDtypeStruct((NND, NREL, FD), jnp.float32),
        mesh=_sc_mesh(),
        compiler_params=pltpu.CompilerParams(use_tc_tiling_on_sc=False),
        scratch_types=[
            *_SPMM_SCRATCH,
            pltpu.VMEM_SHARED((NND, FD), jnp.float32),
            pltpu.SemaphoreType.DMA,
            pltpu.SemaphoreType.DMA,
            pltpu.SemaphoreType.DMA,
        ],
    )


@functools.lru_cache(maxsize=None)
def _make_spmm2():
    return pl.kernel(
        _spmm2_body,
        out_type=jax.ShapeDtypeStruct((NC, NND, FD), jnp.float32),
        mesh=_sc_mesh(),
        compiler_params=pltpu.CompilerParams(use_tc_tiling_on_sc=False),
        scratch_types=[
            *_SPMM_SCRATCH,
            pltpu.VMEM((RPT, FD), jnp.float32),
            pltpu.VMEM_SHARED((NND, FD), jnp.float32),
            pltpu.SemaphoreType.DMA,
            pltpu.SemaphoreType.DMA,
            pltpu.SemaphoreType.DMA,
        ],
    )


def _mm_body(x_ref, w_ref, o_ref):
    o_ref[...] = jnp.dot(x_ref[...], w_ref[...],
                         preferred_element_type=jnp.float32,
                         precision=lax.Precision.HIGHEST)


def _maxfin_body(p_ref, o_ref):
    o_ref[...] = jnp.maximum(jnp.maximum(p_ref[0], p_ref[1]), 0.0)


def _mm(x, w):
    m, k = x.shape
    n = w.shape[1]
    return pl.pallas_call(
        _mm_body,
        grid=(m // BM,),
        in_specs=[pl.BlockSpec((BM, k), lambda mm_: (mm_, 0)),
                  pl.BlockSpec((k, n), lambda mm_: (0, 0))],
        out_specs=pl.BlockSpec((BM, n), lambda mm_: (mm_, 0)),
        out_shape=jax.ShapeDtypeStruct((m, n), jnp.float32),
    )(x, w)


def _maxfin(p):
    return pl.pallas_call(
        _maxfin_body,
        grid=(NND // BM,),
        in_specs=[pl.BlockSpec((NC, BM, FD), lambda m: (0, m, 0))],
        out_specs=pl.BlockSpec((BM, FD), lambda m: (m, 0)),
        out_shape=jax.ShapeDtypeStruct((NND, FD), jnp.float32),
    )(p)


def kernel(x, adj_indices, adj_values, W1, b1, W2, b2):
    adj_indices = adj_indices.astype(jnp.int32)
    adj_rows = adj_indices[:, 0, :].reshape(NREL, NEDGE // G, G)
    adj_cols = adj_indices[:, 1, :].reshape(NREL, NEDGE // G, G)
    vals = adj_values.astype(jnp.float32)

    W1f = W1.transpose(1, 0, 2).reshape(NFEAT, NREL * FD)
    W2f = W2.transpose(1, 0, 2).reshape(NREL * FD, NREL * FD)

    s1f = _mm(x, W1f)
    h3 = _make_spmm1()(s1f.reshape(NND * NREL, FD), adj_rows, adj_cols,
                       vals, b1)
    s2f = _mm(h3.reshape(NND, NREL * FD), W2f)
    pmax = _make_spmm2()(s2f.reshape(NND * NREL, FD), adj_rows, adj_cols,
                         vals, b2)
    return _maxfin(pmax)


# speedup vs baseline: 24.4854x; 1.0045x over previous
import functools

import jax
import jax.numpy as jnp
from jax import lax
from jax.experimental import pallas as pl
from jax.experimental.pallas import tpu as pltpu
from jax.experimental.pallas import tpu_sc as plsc

NREL = 25
NND = 10000
NEDGE = 320000
NFEAT = 128
FD = 16
NC, NS = 2, 16
G = 80
CHUNK = 2000
NG = CHUNK // G
EPT = NEDGE // NS
NCHUNK = EPT // CHUNK
GPT = EPT // G
RPT = NND // NS
RPC = 13
BM = 2000


def _spmm_pipeline(sup_hbm, rows_hbm, cols_hbm, vals_hbm,
                   cols_v, rows_v, vals_v, gath_v, acc_sh,
                   lsem, gsem, ssem, s, i):
    def _fire_loads(j, b):
        goff = s * GPT + j * NG
        eoff = pl.multiple_of(s * EPT + j * CHUNK, 8)
        pltpu.async_copy(cols_hbm.at[i, pl.ds(goff, NG), :], cols_v.at[b],
                         lsem)
        pltpu.async_copy(rows_hbm.at[i, pl.ds(goff, NG), :], rows_v.at[b],
                         lsem)
        pltpu.async_copy(vals_hbm.at[i, pl.ds(eoff, CHUNK)], vals_v.at[b],
                         lsem)

    def _drain_loads(j, b):
        goff = s * GPT + j * NG
        eoff = pl.multiple_of(s * EPT + j * CHUNK, 8)
        pltpu.make_async_copy(cols_hbm.at[i, pl.ds(goff, NG), :],
                              cols_v.at[b], lsem).wait()
        pltpu.make_async_copy(rows_hbm.at[i, pl.ds(goff, NG), :],
                              rows_v.at[b], lsem).wait()
        pltpu.make_async_copy(vals_hbm.at[i, pl.ds(eoff, CHUNK)],
                              vals_v.at[b], lsem).wait()

    def _addoff(b):
        def _body(g, carry):
            for t in range(G // 16):
                sl = pl.ds(t * 16, 16)
                cols_v[b, g, sl] = cols_v[b, g, sl] * NREL + i
            return carry
        lax.fori_loop(0, NG, _body, 0, unroll=2)

    def _fire_gathers(b):
        def _body(g, carry):
            gdst = pl.ds(pl.multiple_of(g * G, 8), G)
            pltpu.async_copy(sup_hbm.at[cols_v.at[b, g]],
                             gath_v.at[b].at[gdst], gsem)
            return carry
        lax.fori_loop(0, NG, _body, 0)

    def _drain_gathers(b):
        def _body(g, carry):
            gdst = pl.ds(pl.multiple_of(g * G, 8), G)
            pltpu.make_async_copy(sup_hbm.at[cols_v.at[b, g]],
                                  gath_v.at[b].at[gdst], gsem).wait()
            return carry
        lax.fori_loop(0, NG, _body, 0)

    def _scale(b):
        @plsc.parallel_loop(0, CHUNK, step=16)
        def _body(base):
            vv = vals_v[b, pl.ds(base, 16)]
            for t in range(16):
                e = base + t
                gath_v[b, e, :] = gath_v[b, e, :] * vv[t]

    def _fire_scatters(b):
        def _body(g, carry):
            gsrc = pl.ds(pl.multiple_of(g * G, 8), G)
            pltpu.async_copy(gath_v.at[b].at[gsrc],
                             acc_sh.at[rows_v.at[b, g]], ssem, add=True)
            return carry
        lax.fori_loop(0, NG, _body, 0)

    def _drain_scatters(b):
        def _body(g, carry):
            gsrc = pl.ds(pl.multiple_of(g * G, 8), G)
            pltpu.make_async_copy(gath_v.at[b].at[gsrc],
                                  acc_sh.at[rows_v.at[b, g]], ssem).wait()
            return carry
        lax.fori_loop(0, NG, _body, 0)

    _fire_loads(0, 0)
    _drain_loads(0, 0)
    _addoff(0)
    _fire_gathers(0)
    for j in range(NCHUNK):
        b = j % 2
        if j + 1 < NCHUNK:
            if j >= 1:
                _drain_scatters(1 - b)
            _fire_loads(j + 1, 1 - b)
        _drain_gathers(b)
        if j + 1 < NCHUNK:
            _drain_loads(j + 1, 1 - b)
            _addoff(1 - b)
            _fire_gathers(1 - b)
        _scale(b)
        _fire_scatters(b)
    _drain_scatters(0)
    _drain_scatters(1)


def _zero_buf(zero_v):
    @plsc.parallel_loop(0, RPT)
    def _zb(r):
        zero_v[r, :] = jnp.zeros((FD,), jnp.float32)


def _spmm1_body(sup_hbm, rows_hbm, cols_hbm, vals_hbm, b_hbm, h_hbm,
                cols_v, rows_v, vals_v, gath_v, zero_v, tmp_v, bv_v,
                acc_sh, lsem, gsem, ssem):
    c = lax.axis_index("c")
    s = lax.axis_index("s")
    rbase = s * RPT
    _zero_buf(zero_v)
    pltpu.sync_copy(b_hbm, bv_v)

    def _rel(k, carry):
        i = c * RPC + k

        @pl.when(i < NREL)
        def _():
            pltpu.sync_copy(zero_v, acc_sh.at[pl.ds(rbase, RPT)])
            plsc.subcore_barrier()
            _spmm_pipeline(sup_hbm, rows_hbm, cols_hbm, vals_hbm,
                           cols_v, rows_v, vals_v, gath_v, acc_sh,
                           lsem, gsem, ssem, s, i)
            plsc.subcore_barrier()
            pltpu.sync_copy(acc_sh.at[pl.ds(rbase, RPT)], tmp_v)
            bv = bv_v[i, :]

            @plsc.parallel_loop(0, RPT)
            def _br(r):
                tmp_v[r, :] = jnp.maximum(tmp_v[r, :] + bv, 0.0)
            pltpu.sync_copy(tmp_v, h_hbm.at[pl.ds(rbase, RPT), i, :])
        return carry
    lax.fori_loop(0, RPC, _rel, 0)


def _spmm2_body(sup_hbm, rows_hbm, cols_hbm, vals_hbm, b_hbm, pmax_hbm,
                cols_v, rows_v, vals_v, gath_v, zero_v, tmp_v, bv_v,
                runmax_v, acc_sh, lsem, gsem, ssem):
    c = lax.axis_index("c")
    s = lax.axis_index("s")
    rbase = s * RPT
    _zero_buf(zero_v)
    pltpu.sync_copy(b_hbm, bv_v)

    def _rel(k, carry):
        i = c * RPC + k

        @pl.when(i < NREL)
        def _():
            pltpu.sync_copy(zero_v, acc_sh.at[pl.ds(rbase, RPT)])
            plsc.subcore_barrier()
            _spmm_pipeline(sup_hbm, rows_hbm, cols_hbm, vals_hbm,
                           cols_v, rows_v, vals_v, gath_v, acc_sh,
                           lsem, gsem, ssem, s, i)
            plsc.subcore_barrier()
            pltpu.sync_copy(acc_sh.at[pl.ds(rbase, RPT)], tmp_v)
            bv = bv_v[i, :]

            @pl.when(k == 0)
            def _():
                @plsc.parallel_loop(0, RPT)
                def _init(r):
                    runmax_v[r, :] = tmp_v[r, :] + bv

            @pl.when(k > 0)
            def _():
                @plsc.parallel_loop(0, RPT)
                def _merge(r):
                    runmax_v[r, :] = jnp.maximum(runmax_v[r, :],
                                                 tmp_v[r, :] + bv)
        return carry
    lax.fori_loop(0, RPC, _rel, 0)
    pltpu.sync_copy(runmax_v, pmax_hbm.at[c, pl.ds(rbase, RPT), :])


_SPMM_SCRATCH = (
    pltpu.VMEM((2, NG, G), jnp.int32),
    pltpu.VMEM((2, NG, G), jnp.int32),
    pltpu.VMEM((2, CHUNK), jnp.float32),
    pltpu.VMEM((2, CHUNK, FD), jnp.float32),
    pltpu.VMEM((RPT, FD), jnp.float32),
    pltpu.VMEM((RPT, FD), jnp.float32),
    pltpu.VMEM((NREL, FD), jnp.float32),
)


def _sc_mesh():
    return plsc.VectorSubcoreMesh(core_axis_name="c", subcore_axis_name="s",
                                  num_cores=NC, num_subcores=NS)


@functools.lru_cache(maxsize=None)
def _make_spmm1():
    return pl.kernel(
        _spmm1_body,
        out_type=jax.ShapeDtypeStruct((NND, NREL, FD), jnp.float32),
        mesh=_sc_mesh(),
        compiler_params=pltpu.CompilerParams(use_tc_tiling_on_sc=False),
        scratch_types=[
            *_SPMM_SCRATCH,
            pltpu.VMEM_SHARED((NND, FD), jnp.float32),
            pltpu.SemaphoreType.DMA,
            pltpu.SemaphoreType.DMA,
            pltpu.SemaphoreType.DMA,
        ],
    )


@functools.lru_cache(maxsize=None)
def _make_spmm2():
    return pl.kernel(
        _spmm2_body,
        out_type=jax.ShapeDtypeStruct((NC, NND, FD), jnp.float32),
        mesh=_sc_mesh(),
        compiler_params=pltpu.CompilerParams(use_tc_tiling_on_sc=False),
        scratch_types=[
            *_SPMM_SCRATCH,
            pltpu.VMEM((RPT, FD), jnp.float32),
            pltpu.VMEM_SHARED((NND, FD), jnp.float32),
            pltpu.SemaphoreType.DMA,
            pltpu.SemaphoreType.DMA,
            pltpu.SemaphoreType.DMA,
        ],
    )


def _mm_body(x_ref, w_ref, o_ref):
    o_ref[...] = jnp.dot(x_ref[...], w_ref[...],
                         preferred_element_type=jnp.float32,
                         precision=lax.Precision.HIGHEST)


def _maxfin_body(p_ref, o_ref):
    o_ref[...] = jnp.maximum(jnp.maximum(p_ref[0], p_ref[1]), 0.0)


def _mm(x, w):
    m, k = x.shape
    n = w.shape[1]
    return pl.pallas_call(
        _mm_body,
        grid=(m // BM,),
        in_specs=[pl.BlockSpec((BM, k), lambda mm_: (mm_, 0)),
                  pl.BlockSpec((k, n), lambda mm_: (0, 0))],
        out_specs=pl.BlockSpec((BM, n), lambda mm_: (mm_, 0)),
        out_shape=jax.ShapeDtypeStruct((m, n), jnp.float32),
    )(x, w)


def _maxfin(p):
    return pl.pallas_call(
        _maxfin_body,
        grid=(NND // BM,),
        in_specs=[pl.BlockSpec((NC, BM, FD), lambda m: (0, m, 0))],
        out_specs=pl.BlockSpec((BM, FD), lambda m: (m, 0)),
        out_shape=jax.ShapeDtypeStruct((NND, FD), jnp.float32),
    )(p)


def kernel(x, adj_indices, adj_values, W1, b1, W2, b2):
    adj_indices = adj_indices.astype(jnp.int32)
    adj_rows = adj_indices[:, 0, :].reshape(NREL, NEDGE // G, G)
    adj_cols = adj_indices[:, 1, :].reshape(NREL, NEDGE // G, G)
    vals = adj_values.astype(jnp.float32)

    W1f = W1.transpose(1, 0, 2).reshape(NFEAT, NREL * FD)
    W2f = W2.transpose(1, 0, 2).reshape(NREL * FD, NREL * FD)

    s1f = _mm(x, W1f)
    h3 = _make_spmm1()(s1f.reshape(NND * NREL, FD), adj_rows, adj_cols,
                       vals, b1)
    s2f = _mm(h3.reshape(NND, NREL * FD), W2f)
    pmax = _make_spmm2()(s2f.reshape(NND * NREL, FD), adj_rows, adj_cols,
                         vals, b2)
    return _maxfin(pmax)
